# baseline (speedup 1.0000x reference)
import functools
import jax
import jax.numpy as jnp
from jax import lax
from jax.experimental import pallas as pl
from jax.experimental.pallas import tpu as pltpu
from jax.experimental.pallas import tpu_sc as plsc

NUSERS = 30000
NNODES = 50000
D = 64
NEDGES = 800000
B = 4096

NC = 2
NS = 16
NW = NC * NS

HALF = NNODES // NC
ACC_ROWS = 25008
ZROWS = ACC_ROWS // NS

SUB = 64

CHP = 24
NBIGP = 17
PROWS_T = NBIGP * CHP
EPADP = NW * PROWS_T * SUB
CAP_TW = 13824
ECAP = NW * CAP_TW
ECAP_ROWS = ECAP // SUB

KF = 3
NSUB = 12
BPC = NSUB // KF
SROWS2 = ECAP_ROWS // NS
NBIG2 = SROWS2 // NSUB
TB2 = NBIG2 * BPC

BPT = B // NW

_mesh = plsc.VectorSubcoreMesh(core_axis_name="c", subcore_axis_name="s")

_GATHER_DN = lax.GatherDimensionNumbers(
    offset_dims=(), collapsed_slice_dims=(0,), start_index_map=(0,))


def _bcast_lane(v16, lane):
    return lax.gather(v16, jnp.full((16, 1), lane, jnp.int32), _GATHER_DN,
                      slice_sizes=(1,),
                      mode=lax.GatherScatterMode.PROMISE_IN_BOUNDS)


def _shuffle(v16, idx16):
    return lax.gather(v16, idx16[:, None], _GATHER_DN, slice_sizes=(1,),
                      mode=lax.GatherScatterMode.PROMISE_IN_BOUNDS)


def _lane_reduce_sum(v16, lanes):
    for sh in (8, 4, 2, 1):
        v16 = v16 + _shuffle(v16, lanes ^ sh)
    return v16


@functools.partial(
    pl.kernel,
    out_type=[
        jax.ShapeDtypeStruct((2, ECAP), jnp.int32),
        jax.ShapeDtypeStruct((2, ECAP), jnp.float32),
        jax.ShapeDtypeStruct((2, ECAP), jnp.int32),
    ],
    mesh=_mesh,
    compiler_params=pltpu.CompilerParams(use_tc_tiling_on_sc=False,
                                         needs_layout_passes=False),
    scratch_types=[
        pltpu.VMEM((2, CHP, SUB), jnp.int32),
        pltpu.VMEM((2, CHP, SUB), jnp.int32),
        pltpu.VMEM((2, CHP, SUB), jnp.float32),
        pltpu.VMEM((CAP_TW,), jnp.int32),
        pltpu.VMEM((CAP_TW,), jnp.int32),
        pltpu.VMEM((CAP_TW,), jnp.float32),
        pltpu.VMEM((CAP_TW,), jnp.float32),
        pltpu.VMEM((CAP_TW,), jnp.int32),
        pltpu.VMEM((CAP_TW,), jnp.int32),
        pltpu.SemaphoreType.DMA,
    ],
)
def _partition(row_hbm, col_hbm, val_hbm, colp_hbm, valp_hbm, lrowp_hbm,
               row_v, col_v, val_v, oc0, oc1, ov0, ov1, olr0, olr1, sem_st):
    c = lax.axis_index("c")
    s = lax.axis_index("s")
    w = s * NC + c
    lanes = lax.iota(jnp.int32, 16)
    izero = jnp.zeros((16,), jnp.int32)
    fzero = jnp.zeros((16,), jnp.float32)

    def zi(i, carry):
        oc0[pl.ds(i * 16, 16)] = izero
        oc1[pl.ds(i * 16, 16)] = izero
        ov0[pl.ds(i * 16, 16)] = fzero
        ov1[pl.ds(i * 16, 16)] = fzero
        olr0[pl.ds(i * 16, 16)] = izero
        olr1[pl.ds(i * 16, 16)] = izero
        return carry
    lax.fori_loop(0, CAP_TW // 16, zi, 0)

    sbase = w * PROWS_T

    def stage_fire(ck, parity):
        off = sbase + ck * CHP
        pltpu.async_copy(row_hbm.at[pl.ds(off, CHP)], row_v.at[parity],
                         sem_st)
        pltpu.async_copy(col_hbm.at[pl.ds(off, CHP)], col_v.at[parity],
                         sem_st)
        pltpu.async_copy(val_hbm.at[pl.ds(off, CHP)], val_v.at[parity],
                         sem_st)

    def stage_drain():
        pltpu.make_async_copy(row_hbm.at[pl.ds(0, CHP)], row_v.at[0],
                              sem_st).wait()
        pltpu.make_async_copy(col_hbm.at[pl.ds(0, CHP)], col_v.at[0],
                              sem_st).wait()
        pltpu.make_async_copy(val_hbm.at[pl.ds(0, CHP)], val_v.at[0],
                              sem_st).wait()

    stage_fire(0, 0)

    def chunk_iter(ck, bases):
        parity = lax.rem(ck, 2)
        stage_drain()

        @pl.when(ck + 1 <= NBIGP - 1)
        def _pf():
            stage_fire(ck + 1, lax.rem(ck + 1, 2))

        def row_iter(r, bases2):
            b0, b1 = bases2
            for q in range(SUB // 16):
                rv = row_v[parity, r, pl.ds(q * 16, 16)]
                cv = col_v[parity, r, pl.ds(q * 16, 16)]
                vv = val_v[parity, r, pl.ds(q * 16, 16)]
                ok0 = rv < HALF
                oki = ok0.astype(jnp.int32)
                ps0 = oki
                for sh in (1, 2, 4, 8):
                    t = _shuffle(ps0, jnp.maximum(lanes - sh, 0))
                    ps0 = jnp.where(lanes >= sh, ps0 + t, ps0)
                ps1 = (lanes + 1) - ps0
                d0 = jnp.minimum(b0 + (ps0 - oki), CAP_TW - 1)
                d1 = jnp.minimum(b1 + (ps1 - (1 - oki)), CAP_TW - 1)
                ok1 = jnp.logical_not(ok0)
                plsc.store_scatter(oc0, [d0], cv, mask=ok0)
                plsc.store_scatter(oc1, [d1], cv, mask=ok1)
                plsc.store_scatter(ov0, [d0], vv, mask=ok0)
                plsc.store_scatter(ov1, [d1], vv, mask=ok1)
                plsc.store_scatter(olr0, [d0], rv, mask=ok0)
                plsc.store_scatter(olr1, [d1], rv - HALF, mask=ok1)
                cnt0 = _bcast_lane(ps0, 15)
                b0 = b0 + cnt0
                b1 = b1 + (16 - cnt0)
            return (b0, b1)
        return lax.fori_loop(0, CHP, row_iter, bases)

    lax.fori_loop(0, NBIGP, chunk_iter, (izero, izero))

    obase = w * CAP_TW
    pltpu.sync_copy(oc0, colp_hbm.at[0, pl.ds(obase, CAP_TW)])
    pltpu.sync_copy(oc1, colp_hbm.at[1, pl.ds(obase, CAP_TW)])
    pltpu.sync_copy(ov0, valp_hbm.at[0, pl.ds(obase, CAP_TW)])
    pltpu.sync_copy(ov1, valp_hbm.at[1, pl.ds(obase, CAP_TW)])
    pltpu.sync_copy(olr0, lrowp_hbm.at[0, pl.ds(obase, CAP_TW)])
    pltpu.sync_copy(olr1, lrowp_hbm.at[1, pl.ds(obase, CAP_TW)])


@functools.partial(
    pl.kernel,
    out_type=jax.ShapeDtypeStruct((NNODES, D), jnp.float32),
    mesh=_mesh,
    compiler_params=pltpu.CompilerParams(use_tc_tiling_on_sc=False),
    scratch_types=[
        pltpu.VMEM((2, NSUB, SUB), jnp.int32),
        pltpu.VMEM((2, NSUB, SUB), jnp.float32),
        pltpu.VMEM((2, NSUB, SUB), jnp.int32),
        pltpu.VMEM((2 * KF, SUB, D), jnp.float32),
        pltpu.VMEM_SHARED((ACC_ROWS, D), jnp.float32),
        pltpu.SemaphoreType.DMA,
        pltpu.SemaphoreType.DMA,
        pltpu.SemaphoreType.DMA,
    ],
)
def _spmm(x_hbm, colp_hbm, valp_hbm, lrowp_hbm, zeros_hbm, y_hbm,
          col_v, val_v, lrow_v, g_v, acc, sem_g, sem_s, sem_st):
    c = lax.axis_index("c")
    s = lax.axis_index("s")
    row_off = c * HALF

    pltpu.sync_copy(zeros_hbm, acc.at[pl.ds(s * ZROWS, ZROWS)])
    plsc.subcore_barrier()

    sbase = s * SROWS2

    def stage_fire(ck, parity):
        off = sbase + ck * NSUB
        pltpu.async_copy(colp_hbm.at[c, pl.ds(off, NSUB)], col_v.at[parity],
                         sem_st)
        pltpu.async_copy(valp_hbm.at[c, pl.ds(off, NSUB)], val_v.at[parity],
                         sem_st)
        pltpu.async_copy(lrowp_hbm.at[c, pl.ds(off, NSUB)], lrow_v.at[parity],
                         sem_st)

    def stage_drain():
        pltpu.make_async_copy(colp_hbm.at[0, pl.ds(0, NSUB)], col_v.at[0],
                              sem_st).wait()
        pltpu.make_async_copy(valp_hbm.at[0, pl.ds(0, NSUB)], val_v.at[0],
                              sem_st).wait()
        pltpu.make_async_copy(lrowp_hbm.at[0, pl.ds(0, NSUB)], lrow_v.at[0],
                              sem_st).wait()

    def fire_gathers(t, slot_base):
        pcf = lax.rem(t // BPC, 2)
        jo = lax.rem(t, BPC) * KF
        for k in range(KF):
            pltpu.async_copy(x_hbm.at[col_v.at[pcf, jo + k]],
                             g_v.at[slot_base + k], sem_g)

    def drain_gathers(slot_base):
        for k in range(KF):
            pltpu.make_async_copy(x_hbm.at[pl.ds(0, SUB)],
                                  g_v.at[slot_base + k], sem_g).wait()

    def scale_batch(t, slot_base):
        pc = lax.rem(t // BPC, 2)
        jo = lax.rem(t, BPC) * KF
        for k in range(KF):
            def grp(gi2, carry):
                vv16 = val_v[pc, jo + k, pl.ds(gi2 * 16, 16)]
                for l in range(16):
                    vv = _bcast_lane(vv16, l)
                    e = gi2 * 16 + l
                    for q in range(D // 16):
                        g_v[slot_base + k, e, pl.ds(q * 16, 16)] = (
                            g_v[slot_base + k, e, pl.ds(q * 16, 16)] * vv)
                return carry
            lax.fori_loop(0, SUB // 16, grp, 0)

    def fire_scatters(t, slot_base):
        pc = lax.rem(t // BPC, 2)
        jo = lax.rem(t, BPC) * KF
        for k in range(KF):
            pltpu.async_copy(g_v.at[slot_base + k],
                             acc.at[lrow_v.at[pc, jo + k]], sem_s, add=True)

    def drain_scatters(slot_base):
        for k in range(KF):
            pltpu.make_async_copy(g_v.at[slot_base + k],
                                  acc.at[pl.ds(0, SUB)], sem_s).wait()

    def body(t, slot_x, drain_sc, fire_g):
        slot_y = KF - slot_x
        drain_gathers(slot_x)

        @pl.when((lax.rem(t, BPC) == BPC - 1) & (t < TB2 - 1))
        def _chunk_in():
            stage_drain()

        if drain_sc:
            drain_scatters(slot_y)

        @pl.when(lax.rem(t, BPC) == 0)
        def _chunk_pf():
            cc = t // BPC

            @pl.when(cc + 1 <= NBIG2 - 1)
            def _pf():
                stage_fire(cc + 1, lax.rem(cc + 1, 2))

        if fire_g:
            fire_gathers(t + 1, slot_y)
        scale_batch(t, slot_x)
        fire_scatters(t, slot_x)

    stage_fire(0, 0)
    stage_drain()
    fire_gathers(0, 0)

    body(0, 0, False, True)
    body(1, KF, True, True)

    def pair(tt, carry):
        t0 = tt * 2
        body(t0, 0, True, True)
        body(t0 + 1, KF, True, True)
        return carry
    lax.fori_loop(1, TB2 // 2 - 1, pair, 0)

    body(TB2 - 2, 0, True, True)
    body(TB2 - 1, KF, True, False)
    drain_scatters(KF)

    plsc.subcore_barrier()

    @pl.when(s < NS - 1)
    def _wb():
        pltpu.sync_copy(acc.at[pl.ds(s * ZROWS, ZROWS)],
                        y_hbm.at[pl.ds(row_off + s * ZROWS, ZROWS)])

    @pl.when(s == NS - 1)
    def _wb_last():
        tail = HALF - (NS - 1) * ZROWS
        pltpu.sync_copy(acc.at[pl.ds((NS - 1) * ZROWS, tail)],
                        y_hbm.at[pl.ds(row_off + (NS - 1) * ZROWS, tail)])


@functools.partial(
    pl.kernel,
    out_type=jax.ShapeDtypeStruct((B,), jnp.float32),
    mesh=_mesh,
    compiler_params=pltpu.CompilerParams(use_tc_tiling_on_sc=False),
    scratch_types=[
        pltpu.VMEM((BPT,), jnp.int32),
        pltpu.VMEM((BPT,), jnp.int32),
        pltpu.VMEM((4, BPT, D), jnp.float32),
        pltpu.VMEM((4, BPT, D), jnp.float32),
        pltpu.VMEM((BPT,), jnp.float32),
        pltpu.SemaphoreType.DMA,
    ],
)
def _score(x0, x1, x2, x3, ui_hbm, ii_hbm, out_hbm,
           ub, ib, gu, gi, ob, sem):
    c = lax.axis_index("c")
    s = lax.axis_index("s")
    w = s * NC + c
    base = w * BPT

    pltpu.sync_copy(ui_hbm.at[pl.ds(base, BPT)], ub)
    pltpu.sync_copy(ii_hbm.at[pl.ds(base, BPT)], ib)
    for t, x in enumerate((x0, x1, x2, x3)):
        pltpu.async_copy(x.at[ub], gu.at[t], sem)
        pltpu.async_copy(x.at[ib], gi.at[t], sem)
    for t, x in enumerate((x0, x1, x2, x3)):
        pltpu.make_async_copy(x.at[pl.ds(0, BPT)], gu.at[t], sem).wait()
        pltpu.make_async_copy(x.at[pl.ds(0, BPT)], gi.at[t], sem).wait()

    lanes = lax.iota(jnp.int32, 16)

    def grp(g_idx, carry):
        pack = jnp.zeros((16,), jnp.float32)
        for l in range(16):
            e = g_idx * 16 + l
            acc = jnp.zeros((16,), jnp.float32)
            for q in range(D // 16):
                uq = (gu[0, e, pl.ds(q * 16, 16)] + gu[1, e, pl.ds(q * 16, 16)]
                      + gu[2, e, pl.ds(q * 16, 16)]
                      + gu[3, e, pl.ds(q * 16, 16)])
                iq = (gi[0, e, pl.ds(q * 16, 16)] + gi[1, e, pl.ds(q * 16, 16)]
                      + gi[2, e, pl.ds(q * 16, 16)]
                      + gi[3, e, pl.ds(q * 16, 16)])
                acc = acc + uq * iq
            red = _lane_reduce_sum(acc, lanes) * jnp.float32(1.0 / 16.0)
            pack = jnp.where(lanes == l, red, pack)
        ob[pl.ds(g_idx * 16, 16)] = pack
        return carry
    lax.fori_loop(0, BPT // 16, grp, 0)

    pltpu.sync_copy(ob, out_hbm.at[pl.ds(base, BPT)])


def kernel(batch, A_indices, A_values, user_emb, item_emb):
    x0 = jnp.concatenate([user_emb, item_emb], axis=0)
    pad = EPADP - NEDGES
    prow = jnp.where(jnp.arange(pad, dtype=jnp.int32) % 2 == 0, 0, HALF)
    row = jnp.concatenate([A_indices[0], prow])
    col = jnp.concatenate([A_indices[1], jnp.zeros((pad,), jnp.int32)])
    val = jnp.concatenate([A_values, jnp.zeros((pad,), jnp.float32)])
    row2 = row.reshape(EPADP // SUB, SUB)
    col2 = col.reshape(EPADP // SUB, SUB)
    val2 = val.reshape(EPADP // SUB, SUB)
    zeros = jnp.zeros((ZROWS, D), jnp.float32)

    colp, valp, lrowp = _partition(row2, col2, val2)
    colp = colp.reshape(2, ECAP_ROWS, SUB)
    valp = valp.reshape(2, ECAP_ROWS, SUB)
    lrowp = lrowp.reshape(2, ECAP_ROWS, SUB)

    x1 = _spmm(x0, colp, valp, lrowp, zeros)
    x2 = _spmm(x1, colp, valp, lrowp, zeros)
    x3 = _spmm(x2, colp, valp, lrowp, zeros)

    ui = batch[:, 0]
    ii = batch[:, 1] + NUSERS
    return _score(x0, x1, x2, x3, ui, ii)


# speedup vs baseline: 1.0021x; 1.0021x over previous
import functools
import jax
import jax.numpy as jnp
from jax import lax
from jax.experimental import pallas as pl
from jax.experimental.pallas import tpu as pltpu
from jax.experimental.pallas import tpu_sc as plsc

NUSERS = 30000
NNODES = 50000
D = 64
NEDGES = 800000
B = 4096

NC = 2
NS = 16
NW = NC * NS

HALF = NNODES // NC
ACC_ROWS = 25008
ZROWS = ACC_ROWS // NS

SUB = 64

CHP = 24
NBIGP = 17
PROWS_T = NBIGP * CHP
EPADP = NW * PROWS_T * SUB
CAP_TW = 13824
ECAP = NW * CAP_TW
ECAP_ROWS = ECAP // SUB

KF = 2
NSUB = 12
BPC = NSUB // KF
SROWS2 = ECAP_ROWS // NS
NBIG2 = SROWS2 // NSUB
TB2 = NBIG2 * BPC

BPT = B // NW

_mesh = plsc.VectorSubcoreMesh(core_axis_name="c", subcore_axis_name="s")

_GATHER_DN = lax.GatherDimensionNumbers(
    offset_dims=(), collapsed_slice_dims=(0,), start_index_map=(0,))


def _bcast_lane(v16, lane):
    return lax.gather(v16, jnp.full((16, 1), lane, jnp.int32), _GATHER_DN,
                      slice_sizes=(1,),
                      mode=lax.GatherScatterMode.PROMISE_IN_BOUNDS)


def _shuffle(v16, idx16):
    return lax.gather(v16, idx16[:, None], _GATHER_DN, slice_sizes=(1,),
                      mode=lax.GatherScatterMode.PROMISE_IN_BOUNDS)


def _lane_reduce_sum(v16, lanes):
    for sh in (8, 4, 2, 1):
        v16 = v16 + _shuffle(v16, lanes ^ sh)
    return v16


@functools.partial(
    pl.kernel,
    out_type=[
        jax.ShapeDtypeStruct((2, ECAP), jnp.int32),
        jax.ShapeDtypeStruct((2, ECAP), jnp.float32),
        jax.ShapeDtypeStruct((2, ECAP), jnp.int32),
    ],
    mesh=_mesh,
    compiler_params=pltpu.CompilerParams(use_tc_tiling_on_sc=False,
                                         needs_layout_passes=False),
    scratch_types=[
        pltpu.VMEM((2, CHP, SUB), jnp.int32),
        pltpu.VMEM((2, CHP, SUB), jnp.int32),
        pltpu.VMEM((2, CHP, SUB), jnp.float32),
        pltpu.VMEM((CAP_TW,), jnp.int32),
        pltpu.VMEM((CAP_TW,), jnp.int32),
        pltpu.VMEM((CAP_TW,), jnp.float32),
        pltpu.VMEM((CAP_TW,), jnp.float32),
        pltpu.VMEM((CAP_TW,), jnp.int32),
        pltpu.VMEM((CAP_TW,), jnp.int32),
        pltpu.SemaphoreType.DMA,
    ],
)
def _partition(row_hbm, col_hbm, val_hbm, colp_hbm, valp_hbm, lrowp_hbm,
               row_v, col_v, val_v, oc0, oc1, ov0, ov1, olr0, olr1, sem_st):
    c = lax.axis_index("c")
    s = lax.axis_index("s")
    w = s * NC + c
    lanes = lax.iota(jnp.int32, 16)
    izero = jnp.zeros((16,), jnp.int32)
    fzero = jnp.zeros((16,), jnp.float32)

    def zi(i, carry):
        oc0[pl.ds(i * 16, 16)] = izero
        oc1[pl.ds(i * 16, 16)] = izero
        ov0[pl.ds(i * 16, 16)] = fzero
        ov1[pl.ds(i * 16, 16)] = fzero
        olr0[pl.ds(i * 16, 16)] = izero
        olr1[pl.ds(i * 16, 16)] = izero
        return carry
    lax.fori_loop(0, CAP_TW // 16, zi, 0)

    sbase = w * PROWS_T

    def stage_fire(ck, parity):
        off = sbase + ck * CHP
        pltpu.async_copy(row_hbm.at[pl.ds(off, CHP)], row_v.at[parity],
                         sem_st)
        pltpu.async_copy(col_hbm.at[pl.ds(off, CHP)], col_v.at[parity],
                         sem_st)
        pltpu.async_copy(val_hbm.at[pl.ds(off, CHP)], val_v.at[parity],
                         sem_st)

    def stage_drain():
        pltpu.make_async_copy(row_hbm.at[pl.ds(0, CHP)], row_v.at[0],
                              sem_st).wait()
        pltpu.make_async_copy(col_hbm.at[pl.ds(0, CHP)], col_v.at[0],
                              sem_st).wait()
        pltpu.make_async_copy(val_hbm.at[pl.ds(0, CHP)], val_v.at[0],
                              sem_st).wait()

    stage_fire(0, 0)

    def chunk_iter(ck, bases):
        parity = lax.rem(ck, 2)
        stage_drain()

        @pl.when(ck + 1 <= NBIGP - 1)
        def _pf():
            stage_fire(ck + 1, lax.rem(ck + 1, 2))

        def row_iter(r, bases2):
            b0, b1 = bases2
            for q in range(SUB // 16):
                rv = row_v[parity, r, pl.ds(q * 16, 16)]
                cv = col_v[parity, r, pl.ds(q * 16, 16)]
                vv = val_v[parity, r, pl.ds(q * 16, 16)]
                ok0 = rv < HALF
                oki = ok0.astype(jnp.int32)
                ps0 = oki
                for sh in (1, 2, 4, 8):
                    t = _shuffle(ps0, jnp.maximum(lanes - sh, 0))
                    ps0 = jnp.where(lanes >= sh, ps0 + t, ps0)
                ps1 = (lanes + 1) - ps0
                d0 = jnp.minimum(b0 + (ps0 - oki), CAP_TW - 1)
                d1 = jnp.minimum(b1 + (ps1 - (1 - oki)), CAP_TW - 1)
                ok1 = jnp.logical_not(ok0)
                plsc.store_scatter(oc0, [d0], cv, mask=ok0)
                plsc.store_scatter(oc1, [d1], cv, mask=ok1)
                plsc.store_scatter(ov0, [d0], vv, mask=ok0)
                plsc.store_scatter(ov1, [d1], vv, mask=ok1)
                plsc.store_scatter(olr0, [d0], rv, mask=ok0)
                plsc.store_scatter(olr1, [d1], rv - HALF, mask=ok1)
                cnt0 = _bcast_lane(ps0, 15)
                b0 = b0 + cnt0
                b1 = b1 + (16 - cnt0)
            return (b0, b1)
        return lax.fori_loop(0, CHP, row_iter, bases)

    lax.fori_loop(0, NBIGP, chunk_iter, (izero, izero))

    obase = w * CAP_TW
    pltpu.sync_copy(oc0, colp_hbm.at[0, pl.ds(obase, CAP_TW)])
    pltpu.sync_copy(oc1, colp_hbm.at[1, pl.ds(obase, CAP_TW)])
    pltpu.sync_copy(ov0, valp_hbm.at[0, pl.ds(obase, CAP_TW)])
    pltpu.sync_copy(ov1, valp_hbm.at[1, pl.ds(obase, CAP_TW)])
    pltpu.sync_copy(olr0, lrowp_hbm.at[0, pl.ds(obase, CAP_TW)])
    pltpu.sync_copy(olr1, lrowp_hbm.at[1, pl.ds(obase, CAP_TW)])


@functools.partial(
    pl.kernel,
    out_type=jax.ShapeDtypeStruct((NNODES, D), jnp.float32),
    mesh=_mesh,
    compiler_params=pltpu.CompilerParams(use_tc_tiling_on_sc=False),
    scratch_types=[
        pltpu.VMEM((2, NSUB, SUB), jnp.int32),
        pltpu.VMEM((2, NSUB, SUB), jnp.float32),
        pltpu.VMEM((2, NSUB, SUB), jnp.int32),
        pltpu.VMEM((3 * KF, SUB, D), jnp.float32),
        pltpu.VMEM_SHARED((ACC_ROWS, D), jnp.float32),
        pltpu.SemaphoreType.DMA,
        pltpu.SemaphoreType.DMA,
        pltpu.SemaphoreType.DMA,
    ],
)
def _spmm(x_hbm, colp_hbm, valp_hbm, lrowp_hbm, zeros_hbm, y_hbm,
          col_v, val_v, lrow_v, g_v, acc, sem_g, sem_s, sem_st):
    c = lax.axis_index("c")
    s = lax.axis_index("s")
    row_off = c * HALF

    pltpu.sync_copy(zeros_hbm, acc.at[pl.ds(s * ZROWS, ZROWS)])
    plsc.subcore_barrier()

    sbase = s * SROWS2

    def stage_fire(ck, parity):
        off = sbase + ck * NSUB
        pltpu.async_copy(colp_hbm.at[c, pl.ds(off, NSUB)], col_v.at[parity],
                         sem_st)
        pltpu.async_copy(valp_hbm.at[c, pl.ds(off, NSUB)], val_v.at[parity],
                         sem_st)
        pltpu.async_copy(lrowp_hbm.at[c, pl.ds(off, NSUB)], lrow_v.at[parity],
                         sem_st)

    def stage_drain():
        pltpu.make_async_copy(colp_hbm.at[0, pl.ds(0, NSUB)], col_v.at[0],
                              sem_st).wait()
        pltpu.make_async_copy(valp_hbm.at[0, pl.ds(0, NSUB)], val_v.at[0],
                              sem_st).wait()
        pltpu.make_async_copy(lrowp_hbm.at[0, pl.ds(0, NSUB)], lrow_v.at[0],
                              sem_st).wait()

    def fire_gathers(t, slot_base):
        pcf = lax.rem(t // BPC, 2)
        jo = lax.rem(t, BPC) * KF
        for k in range(KF):
            pltpu.async_copy(x_hbm.at[col_v.at[pcf, jo + k]],
                             g_v.at[slot_base + k], sem_g)

    def drain_gathers(slot_base):
        for k in range(KF):
            pltpu.make_async_copy(x_hbm.at[pl.ds(0, SUB)],
                                  g_v.at[slot_base + k], sem_g).wait()

    def scale_batch(t, slot_base):
        pc = lax.rem(t // BPC, 2)
        jo = lax.rem(t, BPC) * KF
        for k in range(KF):
            def grp(gi2, carry):
                vv16 = val_v[pc, jo + k, pl.ds(gi2 * 16, 16)]
                for l in range(16):
                    vv = _bcast_lane(vv16, l)
                    e = gi2 * 16 + l
                    for q in range(D // 16):
                        g_v[slot_base + k, e, pl.ds(q * 16, 16)] = (
                            g_v[slot_base + k, e, pl.ds(q * 16, 16)] * vv)
                return carry
            lax.fori_loop(0, SUB // 16, grp, 0)

    def fire_scatters(t, slot_base):
        pc = lax.rem(t // BPC, 2)
        jo = lax.rem(t, BPC) * KF
        for k in range(KF):
            pltpu.async_copy(g_v.at[slot_base + k],
                             acc.at[lrow_v.at[pc, jo + k]], sem_s, add=True)

    def drain_scatters(slot_base):
        for k in range(KF):
            pltpu.make_async_copy(g_v.at[slot_base + k],
                                  acc.at[pl.ds(0, SUB)], sem_s).wait()

    def body(t, set_x, set_y, drain_sc, fire_g):
        drain_gathers(set_x)

        @pl.when((lax.rem(t, BPC) == BPC - 1) & (t < TB2 - 1))
        def _chunk_in():
            stage_drain()

        if drain_sc:
            drain_scatters(set_y)

        @pl.when(lax.rem(t, BPC) == 1)
        def _chunk_pf():
            cc = t // BPC

            @pl.when(cc + 1 <= NBIG2 - 1)
            def _pf():
                stage_fire(cc + 1, lax.rem(cc + 1, 2))

        if fire_g:
            fire_gathers(t + 1, set_y)
        scale_batch(t, set_x)
        fire_scatters(t, set_x)

    S0, S1, S2 = 0, KF, 2 * KF

    stage_fire(0, 0)
    stage_drain()
    fire_gathers(0, S0)

    body(0, S0, S1, False, True)
    body(1, S1, S2, False, True)
    body(2, S2, S0, True, True)

    def triple(tt, carry):
        t0 = tt * 3
        body(t0, S0, S1, True, True)
        body(t0 + 1, S1, S2, True, True)
        body(t0 + 2, S2, S0, True, True)
        return carry
    lax.fori_loop(1, TB2 // 3 - 1, triple, 0)

    body(TB2 - 3, S0, S1, True, True)
    body(TB2 - 2, S1, S2, True, True)
    body(TB2 - 1, S2, S0, True, False)
    drain_scatters(S1)
    drain_scatters(S2)

    plsc.subcore_barrier()

    @pl.when(s < NS - 1)
    def _wb():
        pltpu.sync_copy(acc.at[pl.ds(s * ZROWS, ZROWS)],
                        y_hbm.at[pl.ds(row_off + s * ZROWS, ZROWS)])

    @pl.when(s == NS - 1)
    def _wb_last():
        tail = HALF - (NS - 1) * ZROWS
        pltpu.sync_copy(acc.at[pl.ds((NS - 1) * ZROWS, tail)],
                        y_hbm.at[pl.ds(row_off + (NS - 1) * ZROWS, tail)])


@functools.partial(
    pl.kernel,
    out_type=jax.ShapeDtypeStruct((B,), jnp.float32),
    mesh=_mesh,
    compiler_params=pltpu.CompilerParams(use_tc_tiling_on_sc=False),
    scratch_types=[
        pltpu.VMEM((BPT,), jnp.int32),
        pltpu.VMEM((BPT,), jnp.int32),
        pltpu.VMEM((4, BPT, D), jnp.float32),
        pltpu.VMEM((4, BPT, D), jnp.float32),
        pltpu.VMEM((BPT,), jnp.float32),
        pltpu.SemaphoreType.DMA,
    ],
)
def _score(x0, x1, x2, x3, ui_hbm, ii_hbm, out_hbm,
           ub, ib, gu, gi, ob, sem):
    c = lax.axis_index("c")
    s = lax.axis_index("s")
    w = s * NC + c
    base = w * BPT

    pltpu.sync_copy(ui_hbm.at[pl.ds(base, BPT)], ub)
    pltpu.sync_copy(ii_hbm.at[pl.ds(base, BPT)], ib)
    for t, x in enumerate((x0, x1, x2, x3)):
        pltpu.async_copy(x.at[ub], gu.at[t], sem)
        pltpu.async_copy(x.at[ib], gi.at[t], sem)
    for t, x in enumerate((x0, x1, x2, x3)):
        pltpu.make_async_copy(x.at[pl.ds(0, BPT)], gu.at[t], sem).wait()
        pltpu.make_async_copy(x.at[pl.ds(0, BPT)], gi.at[t], sem).wait()

    lanes = lax.iota(jnp.int32, 16)

    def grp(g_idx, carry):
        pack = jnp.zeros((16,), jnp.float32)
        for l in range(16):
            e = g_idx * 16 + l
            acc = jnp.zeros((16,), jnp.float32)
            for q in range(D // 16):
                uq = (gu[0, e, pl.ds(q * 16, 16)] + gu[1, e, pl.ds(q * 16, 16)]
                      + gu[2, e, pl.ds(q * 16, 16)]
                      + gu[3, e, pl.ds(q * 16, 16)])
                iq = (gi[0, e, pl.ds(q * 16, 16)] + gi[1, e, pl.ds(q * 16, 16)]
                      + gi[2, e, pl.ds(q * 16, 16)]
                      + gi[3, e, pl.ds(q * 16, 16)])
                acc = acc + uq * iq
            red = _lane_reduce_sum(acc, lanes) * jnp.float32(1.0 / 16.0)
            pack = jnp.where(lanes == l, red, pack)
        ob[pl.ds(g_idx * 16, 16)] = pack
        return carry
    lax.fori_loop(0, BPT // 16, grp, 0)

    pltpu.sync_copy(ob, out_hbm.at[pl.ds(base, BPT)])


def kernel(batch, A_indices, A_values, user_emb, item_emb):
    x0 = jnp.concatenate([user_emb, item_emb], axis=0)
    pad = EPADP - NEDGES
    prow = jnp.where(jnp.arange(pad, dtype=jnp.int32) % 2 == 0, 0, HALF)
    row = jnp.concatenate([A_indices[0], prow])
    col = jnp.concatenate([A_indices[1], jnp.zeros((pad,), jnp.int32)])
    val = jnp.concatenate([A_values, jnp.zeros((pad,), jnp.float32)])
    row2 = row.reshape(EPADP // SUB, SUB)
    col2 = col.reshape(EPADP // SUB, SUB)
    val2 = val.reshape(EPADP // SUB, SUB)
    zeros = jnp.zeros((ZROWS, D), jnp.float32)

    colp, valp, lrowp = _partition(row2, col2, val2)
    colp = colp.reshape(2, ECAP_ROWS, SUB)
    valp = valp.reshape(2, ECAP_ROWS, SUB)
    lrowp = lrowp.reshape(2, ECAP_ROWS, SUB)

    x1 = _spmm(x0, colp, valp, lrowp, zeros)
    x2 = _spmm(x1, colp, valp, lrowp, zeros)
    x3 = _spmm(x2, colp, valp, lrowp, zeros)

    ui = batch[:, 0]
    ii = batch[:, 1] + NUSERS
    return _score(x0, x1, x2, x3, ui, ii)
